# baseline (speedup 1.0000x reference)
import functools

import jax
import jax.numpy as jnp
from jax.experimental import pallas as pl
from jax.experimental.pallas import tpu as pltpu

PATCH = 7
BLK_I = 1024
BLK_T = 1024


def _body(x_ref, yt_ref, out_ref, min_ref, acc_ref, *, n_real, nt, ni, nb, scale):
    b = pl.program_id(0)
    i = pl.program_id(1)
    t = pl.program_id(2)

    @pl.when(t == 0)
    def _init_min():
        min_ref[...] = jnp.full_like(min_ref, jnp.inf)

    @pl.when((b == 0) & (i == 0) & (t == 0))
    def _init_acc():
        acc_ref[0, 0] = jnp.float32(0.0)

    xb = x_ref[0]
    yb = yt_ref[0]
    s = jax.lax.dot_general(
        xb, yb, (((1,), (0,)), ((), ())),
        preferred_element_type=jnp.float32,
        precision=jax.lax.Precision.HIGHEST,
    )
    ny = jnp.sum(yb * yb, axis=0, keepdims=True)
    dist = ny - 2.0 * s
    m = jnp.min(dist, axis=1, keepdims=True)
    min_ref[...] = jnp.minimum(min_ref[...], m)

    @pl.when(t == nt - 1)
    def _accumulate():
        nx = jnp.sum(xb * xb, axis=1, keepdims=True)
        ids = jax.lax.broadcasted_iota(jnp.int32, (min_ref.shape[0], 1), 0) + i * min_ref.shape[0]
        vals = jnp.where(ids < n_real, min_ref[...] + nx, 0.0)
        acc_ref[0, 0] += jnp.sum(vals)

    @pl.when((b == nb - 1) & (i == ni - 1) & (t == nt - 1))
    def _finalize():
        out_ref[...] = jnp.full((1, 1), acc_ref[0, 0] * scale, jnp.float32)


def _patches(img):
    p = PATCH
    pt = jax.lax.conv_general_dilated_patches(
        img, filter_shape=(p, p), window_strides=(1, 1), padding='VALID')
    b, d = pt.shape[0], pt.shape[1]
    return pt.reshape(b, d, -1)


@jax.jit
def kernel(x, y):
    b = x.shape[0]
    xt = _patches(x)
    yt = _patches(y)
    d, n = xt.shape[1], xt.shape[2]

    n_pad = ((n + BLK_I - 1) // BLK_I) * BLK_I
    n_pad = ((n_pad + BLK_T - 1) // BLK_T) * BLK_T
    xp = jnp.pad(jnp.transpose(xt, (0, 2, 1)), ((0, 0), (0, n_pad - n), (0, 0)))
    ypad = jnp.broadcast_to(yt[:, :, 0:1], (b, d, n_pad - n))
    yp = jnp.concatenate([yt, ypad], axis=2)

    ni = n_pad // BLK_I
    nt = n_pad // BLK_T
    scale = 1.0 / (b * n * d)

    body = functools.partial(_body, n_real=n, nt=nt, ni=ni, nb=b, scale=scale)
    out = pl.pallas_call(
        body,
        grid=(b, ni, nt),
        in_specs=[
            pl.BlockSpec((1, BLK_I, d), lambda bi, ii, ti: (bi, ii, 0)),
            pl.BlockSpec((1, d, BLK_T), lambda bi, ii, ti: (bi, 0, ti)),
        ],
        out_specs=pl.BlockSpec((1, 1), lambda bi, ii, ti: (0, 0)),
        out_shape=jax.ShapeDtypeStruct((1, 1), jnp.float32),
        scratch_shapes=[
            pltpu.VMEM((BLK_I, 1), jnp.float32),
            pltpu.SMEM((1, 1), jnp.float32),
        ],
        compiler_params=pltpu.CompilerParams(
            dimension_semantics=("arbitrary", "arbitrary", "arbitrary"),
        ),
    )(xp, yp)
    return out[0, 0]


# speedup vs baseline: 2.0730x; 2.0730x over previous
import functools

import jax
import jax.numpy as jnp
from jax.experimental import pallas as pl
from jax.experimental.pallas import tpu as pltpu

PATCH = 7
BLK_I = 1024
BLK_T = 1024


def _body(x_ref, yt_ref, out_ref, min_ref, acc_ref, *, n_real, nt, ni, nb, scale):
    b = pl.program_id(0)
    i = pl.program_id(1)
    t = pl.program_id(2)

    @pl.when(t == 0)
    def _init_min():
        min_ref[...] = jnp.full_like(min_ref, jnp.inf)

    @pl.when((b == 0) & (i == 0) & (t == 0))
    def _init_acc():
        acc_ref[0, 0] = jnp.float32(0.0)

    xb = x_ref[0]
    yb = yt_ref[0]
    s = jax.lax.dot_general(
        xb.astype(jnp.bfloat16), yb.astype(jnp.bfloat16), (((1,), (0,)), ((), ())),
        preferred_element_type=jnp.float32,
    )
    ny = jnp.sum(yb * yb, axis=0, keepdims=True)
    dist = ny - 2.0 * s
    m = jnp.min(dist, axis=1, keepdims=True)
    min_ref[...] = jnp.minimum(min_ref[...], m)

    @pl.when(t == nt - 1)
    def _accumulate():
        nx = jnp.sum(xb * xb, axis=1, keepdims=True)
        ids = jax.lax.broadcasted_iota(jnp.int32, (min_ref.shape[0], 1), 0) + i * min_ref.shape[0]
        vals = jnp.where(ids < n_real, min_ref[...] + nx, 0.0)
        acc_ref[0, 0] += jnp.sum(vals)

    @pl.when((b == nb - 1) & (i == ni - 1) & (t == nt - 1))
    def _finalize():
        out_ref[...] = jnp.full((1, 1), acc_ref[0, 0] * scale, jnp.float32)


def _patches(img):
    p = PATCH
    pt = jax.lax.conv_general_dilated_patches(
        img, filter_shape=(p, p), window_strides=(1, 1), padding='VALID')
    b, d = pt.shape[0], pt.shape[1]
    return pt.reshape(b, d, -1)


@jax.jit
def kernel(x, y):
    b = x.shape[0]
    xt = _patches(x)
    yt = _patches(y)
    d, n = xt.shape[1], xt.shape[2]

    n_pad = ((n + BLK_I - 1) // BLK_I) * BLK_I
    n_pad = ((n_pad + BLK_T - 1) // BLK_T) * BLK_T
    xp = jnp.pad(jnp.transpose(xt, (0, 2, 1)), ((0, 0), (0, n_pad - n), (0, 0)))
    ypad = jnp.broadcast_to(yt[:, :, 0:1], (b, d, n_pad - n))
    yp = jnp.concatenate([yt, ypad], axis=2)

    ni = n_pad // BLK_I
    nt = n_pad // BLK_T
    scale = 1.0 / (b * n * d)

    body = functools.partial(_body, n_real=n, nt=nt, ni=ni, nb=b, scale=scale)
    out = pl.pallas_call(
        body,
        grid=(b, ni, nt),
        in_specs=[
            pl.BlockSpec((1, BLK_I, d), lambda bi, ii, ti: (bi, ii, 0)),
            pl.BlockSpec((1, d, BLK_T), lambda bi, ii, ti: (bi, 0, ti)),
        ],
        out_specs=pl.BlockSpec((1, 1), lambda bi, ii, ti: (0, 0)),
        out_shape=jax.ShapeDtypeStruct((1, 1), jnp.float32),
        scratch_shapes=[
            pltpu.VMEM((BLK_I, 1), jnp.float32),
            pltpu.SMEM((1, 1), jnp.float32),
        ],
        compiler_params=pltpu.CompilerParams(
            dimension_semantics=("arbitrary", "arbitrary", "arbitrary"),
        ),
    )(xp, yp)
    return out[0, 0]


# speedup vs baseline: 3.4156x; 1.6477x over previous
import functools

import jax
import jax.numpy as jnp
from jax.experimental import pallas as pl
from jax.experimental.pallas import tpu as pltpu

PATCH = 7
BLK = 1024


def _round_up(v, m):
    return ((v + m - 1) // m) * m


def _body(xf_ref, yf_ref, out_ref, sxd, sxn, sy, nyac, nyv, mins, acc,
          *, c, w, d, npad, npos, ni, nt, nb, scale):
    b = pl.program_id(0)
    i = pl.program_id(1)
    t = pl.program_id(2)

    lane = jax.lax.broadcasted_iota(jnp.int32, (1, npad), 1)
    lane_valid = (jnp.remainder(lane, w) < (w - PATCH + 1)) & (lane < npos)

    @pl.when((b == 0) & (i == 0) & (t == 0))
    def _init_acc():
        acc[0, 0] = jnp.float32(0.0)

    @pl.when((i == 0) & (t == 0))
    def _build():
        for dd in range(d):
            ch, rem = divmod(dd, PATCH * PATCH)
            dy, dx = divmod(rem, PATCH)
            off = dy * w + dx
            row = yf_ref[0, pl.ds(ch, 1), pl.ds(off, npad)]
            sy[pl.ds(dd, 1), :] = (row * -2.0).astype(jnp.bfloat16)
            if dd < 8:
                nyac[pl.ds(dd, 1), :] = row * row
            else:
                nyac[pl.ds(dd % 8, 1), :] = nyac[pl.ds(dd % 8, 1), :] + row * row
        nyv[...] = jnp.where(
            lane_valid,
            jnp.sum(nyac[...], axis=0, keepdims=True),
            jnp.float32(1e30))
        for dd in range(d):
            ch, rem = divmod(dd, PATCH * PATCH)
            dy, dx = divmod(rem, PATCH)
            off = dy * w + dx
            row = xf_ref[0, pl.ds(ch, 1), pl.ds(off, npad)]
            sxd[pl.ds(dd, 1), :] = row
            if dd < 8:
                nyac[pl.ds(dd, 1), :] = row * row
            else:
                nyac[pl.ds(dd % 8, 1), :] = nyac[pl.ds(dd % 8, 1), :] + row * row
        nx = jnp.sum(nyac[...], axis=0, keepdims=True)
        acc[0, 0] += jnp.sum(jnp.where(lane_valid, nx, 0.0))
        for ib in range(ni):
            blk = sxd[:, pl.ds(ib * BLK, BLK)]
            sxn[pl.ds(ib * BLK, BLK), :] = jnp.transpose(blk, (1, 0)).astype(jnp.bfloat16)

    @pl.when(t == 0)
    def _init_min():
        mins[...] = jnp.full_like(mins, jnp.inf)

    xb = sxn[pl.ds(pl.multiple_of(i * BLK, BLK), BLK), :]
    yb = sy[:, pl.ds(pl.multiple_of(t * BLK, BLK), BLK)]
    s = jax.lax.dot_general(
        xb, yb, (((1,), (0,)), ((), ())),
        preferred_element_type=jnp.float32)
    nyb = nyv[:, pl.ds(pl.multiple_of(t * BLK, BLK), BLK)]
    m = None
    for k in range(BLK // 128):
        part = s[:, k * 128:(k + 1) * 128] + nyb[:, k * 128:(k + 1) * 128]
        m = part if m is None else jnp.minimum(m, part)
    mins[...] = jnp.minimum(mins[...], m)

    @pl.when(t == nt - 1)
    def _accumulate():
        mrow = jnp.min(mins[...], axis=1, keepdims=True)
        pos = jax.lax.broadcasted_iota(jnp.int32, (BLK, 1), 0) + i * BLK
        valid = (jnp.remainder(pos, w) < (w - PATCH + 1)) & (pos < npos)
        acc[0, 0] += jnp.sum(jnp.where(valid, mrow, 0.0))

    @pl.when((b == nb - 1) & (i == ni - 1) & (t == nt - 1))
    def _finalize():
        out_ref[...] = jnp.full((1, 1), acc[0, 0] * scale, jnp.float32)


@jax.jit
def kernel(x, y):
    b, c, h, w = x.shape
    p = PATCH
    oh, ow = h - p + 1, w - p + 1
    d = c * p * p
    n_real = oh * ow
    npos = oh * w
    npad = _round_up(npos, BLK)
    flatpad = _round_up(npad + (p - 1) * w + p, 128)

    xf = jnp.pad(x.reshape(b, c, h * w), ((0, 0), (0, 0), (0, flatpad - h * w)))
    yf = jnp.pad(y.reshape(b, c, h * w), ((0, 0), (0, 0), (0, flatpad - h * w)))

    ni = npad // BLK
    nt = npad // BLK
    scale = 1.0 / (b * n_real * d)

    body = functools.partial(
        _body, c=c, w=w, d=d, npad=npad, npos=npos,
        ni=ni, nt=nt, nb=b, scale=scale)
    out = pl.pallas_call(
        body,
        grid=(b, ni, nt),
        in_specs=[
            pl.BlockSpec((1, c, flatpad), lambda bi, ii, ti: (bi, 0, 0)),
            pl.BlockSpec((1, c, flatpad), lambda bi, ii, ti: (bi, 0, 0)),
        ],
        out_specs=pl.BlockSpec((1, 1), lambda bi, ii, ti: (0, 0)),
        out_shape=jax.ShapeDtypeStruct((1, 1), jnp.float32),
        scratch_shapes=[
            pltpu.VMEM((d, npad), jnp.float32),
            pltpu.VMEM((npad, d), jnp.bfloat16),
            pltpu.VMEM((d, npad), jnp.bfloat16),
            pltpu.VMEM((8, npad), jnp.float32),
            pltpu.VMEM((1, npad), jnp.float32),
            pltpu.VMEM((BLK, 128), jnp.float32),
            pltpu.SMEM((1, 1), jnp.float32),
        ],
        compiler_params=pltpu.CompilerParams(
            dimension_semantics=("arbitrary", "arbitrary", "arbitrary"),
        ),
    )(xf, yf)
    return out[0, 0]


# speedup vs baseline: 3.4244x; 1.0026x over previous
import functools

import jax
import jax.numpy as jnp
from jax.experimental import pallas as pl
from jax.experimental.pallas import tpu as pltpu

PATCH = 7
BLK = 1024


def _round_up(v, m):
    return ((v + m - 1) // m) * m


def _body(xf_ref, yf_ref, out_ref, sxd, sxn, sy, nyac, mins, acc,
          *, c, w, d, npad, npos, ni, nt, nb, scale):
    b = pl.program_id(0)
    i = pl.program_id(1)
    t = pl.program_id(2)

    lane = jax.lax.broadcasted_iota(jnp.int32, (1, npad), 1)
    lane_valid = (jnp.remainder(lane, w) < (w - PATCH + 1)) & (lane < npos)

    @pl.when((b == 0) & (i == 0) & (t == 0))
    def _init_acc():
        acc[0, 0] = jnp.float32(0.0)

    @pl.when((i == 0) & (t == 0))
    def _build():
        for dd in range(d):
            ch, rem = divmod(dd, PATCH * PATCH)
            dy, dx = divmod(rem, PATCH)
            off = dy * w + dx
            row = yf_ref[0, pl.ds(ch, 1), pl.ds(off, npad)]
            sy[pl.ds(dd, 1), :] = (row * -2.0).astype(jnp.bfloat16)
            if dd < 8:
                nyac[pl.ds(dd, 1), :] = row * row
            else:
                nyac[pl.ds(dd % 8, 1), :] = nyac[pl.ds(dd % 8, 1), :] + row * row
        nyf = jnp.where(
            lane_valid,
            jnp.sum(nyac[...], axis=0, keepdims=True),
            jnp.float32(1e30))
        ny_hi = nyf.astype(jnp.bfloat16)
        ny_lo = (nyf - ny_hi.astype(jnp.float32)).astype(jnp.bfloat16)
        sy[pl.ds(d, 1), :] = ny_hi
        sy[pl.ds(d + 1, 1), :] = ny_lo
        for dd in range(d):
            ch, rem = divmod(dd, PATCH * PATCH)
            dy, dx = divmod(rem, PATCH)
            off = dy * w + dx
            row = xf_ref[0, pl.ds(ch, 1), pl.ds(off, npad)]
            sxd[pl.ds(dd, 1), :] = row
            if dd < 8:
                nyac[pl.ds(dd, 1), :] = row * row
            else:
                nyac[pl.ds(dd % 8, 1), :] = nyac[pl.ds(dd % 8, 1), :] + row * row
        nx = jnp.sum(nyac[...], axis=0, keepdims=True)
        acc[0, 0] += jnp.sum(jnp.where(lane_valid, nx, 0.0))
        for ib in range(ni):
            blk = sxd[:, pl.ds(ib * BLK, BLK)]
            sxn[pl.ds(ib * BLK, BLK), pl.ds(0, d)] = (
                jnp.transpose(blk, (1, 0)).astype(jnp.bfloat16))
        sxn[:, pl.ds(d, 2)] = jnp.ones((npad, 2), jnp.bfloat16)

    @pl.when(t == 0)
    def _init_min():
        mins[...] = jnp.full_like(mins, jnp.inf)

    xb = sxn[pl.ds(pl.multiple_of(i * BLK, BLK), BLK), :]
    yb = sy[:, pl.ds(pl.multiple_of(t * BLK, BLK), BLK)]
    s = jax.lax.dot_general(
        xb, yb, (((1,), (0,)), ((), ())),
        preferred_element_type=jnp.float32)
    m = None
    for k in range(BLK // 128):
        part = s[:, k * 128:(k + 1) * 128]
        m = part if m is None else jnp.minimum(m, part)
    mins[...] = jnp.minimum(mins[...], m)

    @pl.when(t == nt - 1)
    def _accumulate():
        mrow = jnp.min(mins[...], axis=1, keepdims=True)
        pos = jax.lax.broadcasted_iota(jnp.int32, (BLK, 1), 0) + i * BLK
        valid = (jnp.remainder(pos, w) < (w - PATCH + 1)) & (pos < npos)
        acc[0, 0] += jnp.sum(jnp.where(valid, mrow, 0.0))

    @pl.when((b == nb - 1) & (i == ni - 1) & (t == nt - 1))
    def _finalize():
        out_ref[...] = jnp.full((1, 1), acc[0, 0] * scale, jnp.float32)


@jax.jit
def kernel(x, y):
    b, c, h, w = x.shape
    p = PATCH
    oh, ow = h - p + 1, w - p + 1
    d = c * p * p
    n_real = oh * ow
    npos = oh * w
    npad = _round_up(npos, BLK)
    flatpad = _round_up(npad + (p - 1) * w + p, 128)

    xf = jnp.pad(x.reshape(b, c, h * w), ((0, 0), (0, 0), (0, flatpad - h * w)))
    yf = jnp.pad(y.reshape(b, c, h * w), ((0, 0), (0, 0), (0, flatpad - h * w)))

    ni = npad // BLK
    nt = npad // BLK
    scale = 1.0 / (b * n_real * d)

    body = functools.partial(
        _body, c=c, w=w, d=d, npad=npad, npos=npos,
        ni=ni, nt=nt, nb=b, scale=scale)
    out = pl.pallas_call(
        body,
        grid=(b, ni, nt),
        in_specs=[
            pl.BlockSpec((1, c, flatpad), lambda bi, ii, ti: (bi, 0, 0)),
            pl.BlockSpec((1, c, flatpad), lambda bi, ii, ti: (bi, 0, 0)),
        ],
        out_specs=pl.BlockSpec((1, 1), lambda bi, ii, ti: (0, 0)),
        out_shape=jax.ShapeDtypeStruct((1, 1), jnp.float32),
        scratch_shapes=[
            pltpu.VMEM((d, npad), jnp.float32),
            pltpu.VMEM((npad, d + 2), jnp.bfloat16),
            pltpu.VMEM((d + 2, npad), jnp.bfloat16),
            pltpu.VMEM((8, npad), jnp.float32),
            pltpu.VMEM((BLK, 128), jnp.float32),
            pltpu.SMEM((1, 1), jnp.float32),
        ],
        compiler_params=pltpu.CompilerParams(
            dimension_semantics=("arbitrary", "arbitrary", "arbitrary"),
        ),
    )(xf, yf)
    return out[0, 0]
